# baseline (speedup 1.0000x reference)
import functools

import jax
import jax.numpy as jnp
from jax import lax
from jax.experimental import pallas as pl
from jax.experimental.pallas import tpu as pltpu
from jax.experimental.pallas import tpu_sc as plsc

EMBED_DIM = 32
BATCH = 4096
HIST = 200
NUM_CORES = 2
NUM_SUBCORES = 16
NW = NUM_CORES * NUM_SUBCORES
CHUNK = 128
BLOCKS = HIST * (BATCH // CHUNK)
PER_W = BLOCKS // NW
NBUF = 4
NBC = BATCH // CHUNK

_mesh = plsc.VectorSubcoreMesh(core_axis_name="c", subcore_axis_name="s")


@functools.partial(
    pl.kernel,
    mesh=_mesh,
    out_type=jax.ShapeDtypeStruct((HIST, EMBED_DIM, BATCH), jnp.float32),
    scratch_types=[
        pltpu.VMEM((PER_W, CHUNK), jnp.int32),
        pltpu.VMEM((NBUF, CHUNK, EMBED_DIM), jnp.float32),
        pltpu.VMEM((NBUF, EMBED_DIM, CHUNK), jnp.float32),
        [pltpu.SemaphoreType.DMA] * NBUF,
        [pltpu.SemaphoreType.DMA] * NBUF,
    ],
    compiler_params=pltpu.CompilerParams(
        use_tc_tiling_on_sc=False, needs_layout_passes=False
    ),
)
def _emb_gather_t(x_hbm, table_hbm, out_hbm, idx_v, rows_v, tb_v, gsems, wsems):
    w = lax.axis_index("s") * NUM_CORES + lax.axis_index("c")
    g0 = w * PER_W

    pltpu.sync_copy(x_hbm.at[w], idx_v)

    for b in range(NBUF):
        pltpu.async_copy(table_hbm.at[idx_v.at[b]], rows_v.at[b], gsems[b])

    lane = lax.iota(jnp.int32, 16)
    b_idx = [j * 16 + lane for j in range(CHUNK // 16)]

    def body(i, _):
        for b in range(NBUF):
            g = i * NBUF + b
            blk = g0 + g
            h = blk // NBC
            bc = blk - h * NBC
            out_slice = out_hbm.at[h, :, pl.ds(bc * CHUNK, CHUNK)]
            pltpu.make_async_copy(
                table_hbm.at[idx_v.at[g]], rows_v.at[b], gsems[b]
            ).wait()

            @pl.when(g >= NBUF)
            def _drain():
                pltpu.make_async_copy(tb_v.at[b], out_slice, wsems[b]).wait()

            def etrans(e, carry):
                e_vec = jnp.full((16,), e, dtype=jnp.int32)
                for j in range(CHUNK // 16):
                    v = plsc.load_gather(rows_v.at[b], [b_idx[j], e_vec])
                    tb_v[b, e, pl.ds(j * 16, 16)] = v
                return carry

            lax.fori_loop(0, EMBED_DIM, etrans, 0)

            pltpu.async_copy(tb_v.at[b], out_slice, wsems[b])
            nxt = g + NBUF

            @pl.when(nxt < PER_W)
            def _refill():
                pltpu.async_copy(
                    table_hbm.at[idx_v.at[nxt]], rows_v.at[b], gsems[b]
                )

        return _

    lax.fori_loop(0, PER_W // NBUF, body, None)

    for b in range(NBUF):
        g = PER_W - NBUF + b
        blk = g0 + g
        h = blk // NBC
        bc = blk - h * NBC
        pltpu.make_async_copy(
            tb_v.at[b], out_hbm.at[h, :, pl.ds(bc * CHUNK, CHUNK)], wsems[b]
        ).wait()


@jax.jit
def kernel(x, table):
    xt = x.astype(jnp.int32).T.reshape(NW, PER_W, CHUNK)
    out_t = _emb_gather_t(xt, table)
    return out_t.transpose(2, 0, 1)


# speedup vs baseline: 1.3093x; 1.3093x over previous
import functools

import jax
import jax.numpy as jnp
from jax import lax
from jax.experimental import pallas as pl
from jax.experimental.pallas import tpu as pltpu
from jax.experimental.pallas import tpu_sc as plsc

EMBED_DIM = 32
BATCH = 4096
HIST = 200
NUM_CORES = 2
NUM_SUBCORES = 16
NW = NUM_CORES * NUM_SUBCORES
CHUNK = 128
BLOCKS = HIST * (BATCH // CHUNK)
PER_W = BLOCKS // NW
NBUF = 4
NBC = BATCH // CHUNK
BLK = CHUNK * EMBED_DIM
NVREG = BLK // 16
GRP = 32

_mesh = plsc.VectorSubcoreMesh(core_axis_name="c", subcore_axis_name="s")


@functools.partial(
    pl.kernel,
    mesh=_mesh,
    out_type=jax.ShapeDtypeStruct((HIST, EMBED_DIM, BATCH), jnp.float32),
    scratch_types=[
        pltpu.VMEM((PER_W, CHUNK), jnp.int32),
        pltpu.VMEM((NBUF, CHUNK, EMBED_DIM), jnp.float32),
        pltpu.VMEM((NBUF, EMBED_DIM, CHUNK), jnp.float32),
        [pltpu.SemaphoreType.DMA] * NBUF,
        [pltpu.SemaphoreType.DMA] * NBUF,
    ],
    compiler_params=pltpu.CompilerParams(
        use_tc_tiling_on_sc=False, needs_layout_passes=False
    ),
)
def _emb_gather_t(x_hbm, table_hbm, out_hbm, idx_v, rows_v, tb_v, gsems, wsems):
    w = lax.axis_index("s") * NUM_CORES + lax.axis_index("c")
    g0 = w * PER_W

    pltpu.sync_copy(x_hbm.at[w], idx_v)

    for b in range(NBUF):
        pltpu.async_copy(table_hbm.at[idx_v.at[b]], rows_v.at[b], gsems[b])

    lane = lax.iota(jnp.int32, 16)
    row_idx = [lane + j * 16 for j in range(CHUNK // 16)]

    def body(i, _):
        for b in range(NBUF):
            g = i * NBUF + b
            blk = g0 + g
            h = blk // NBC
            bc = blk - h * NBC
            out_slice = out_hbm.at[h, :, pl.ds(bc * CHUNK, CHUNK)]
            pltpu.make_async_copy(
                table_hbm.at[idx_v.at[g]], rows_v.at[b], gsems[b]
            ).wait()

            @pl.when(g >= NBUF)
            def _drain():
                pltpu.make_async_copy(tb_v.at[b], out_slice, wsems[b]).wait()

            @plsc.parallel_loop(0, EMBED_DIM, 1, unroll=4)
            def trans(e):
                e_vec = jnp.full((16,), e, dtype=jnp.int32)
                for j in range(CHUNK // 16):
                    v = plsc.load_gather(rows_v.at[b], [row_idx[j], e_vec])
                    tb_v[b, e, pl.ds(j * 16, 16)] = v

            pltpu.async_copy(tb_v.at[b], out_slice, wsems[b])
            nxt = g + NBUF

            @pl.when(nxt < PER_W)
            def _refill():
                pltpu.async_copy(
                    table_hbm.at[idx_v.at[nxt]], rows_v.at[b], gsems[b]
                )

        return _

    lax.fori_loop(0, PER_W // NBUF, body, None)

    for b in range(NBUF):
        g = PER_W - NBUF + b
        blk = g0 + g
        h = blk // NBC
        bc = blk - h * NBC
        pltpu.make_async_copy(
            tb_v.at[b], out_hbm.at[h, :, pl.ds(bc * CHUNK, CHUNK)], wsems[b]
        ).wait()


@jax.jit
def kernel(x, table):
    xt = x.astype(jnp.int32).T.reshape(NW, PER_W, CHUNK)
    out_t = _emb_gather_t(xt, table)
    return out_t.transpose(2, 0, 1)
